# Initial kernel scaffold; baseline (speedup 1.0000x reference)
import jax
import jax.numpy as jnp
from jax.experimental import pallas as pl


def kernel(x, E_class):
    raise NotImplementedError("write your pallas kernel here")


# speedup vs baseline: 1.3105x; 1.3105x over previous
import functools

import jax
import jax.numpy as jnp
from jax import lax
from jax.experimental import pallas as pl
from jax.experimental.pallas import tpu as pltpu
from jax.experimental.pallas import tpu_sc as plsc

SEQ_LEN = 16384
E_DIMS = 256
NUM_WORKERS = 32
B_PER_W = SEQ_LEN // NUM_WORKERS
CHUNK = 128
NCHUNK = B_PER_W // CHUNK


def _gather_kernel(x_hbm, tbl_hbm, out_hbm, idx_v, rows_v, sem):
    wid = lax.axis_index("s") * 2 + lax.axis_index("c")
    base = wid * B_PER_W
    pltpu.sync_copy(x_hbm.at[wid], idx_v)
    for j in range(NCHUNK):
        pltpu.async_copy(tbl_hbm.at[idx_v.at[j]], rows_v, sem).wait()
        pltpu.sync_copy(rows_v, out_hbm.at[pl.ds(base + j * CHUNK, CHUNK)])


@functools.partial(jax.jit, static_argnames=())
def kernel(x, E_class):
    x32 = x.astype(jnp.int32).reshape(NUM_WORKERS, NCHUNK, CHUNK)
    mesh = plsc.VectorSubcoreMesh(core_axis_name="c", subcore_axis_name="s")
    k = functools.partial(
        pl.kernel,
        mesh=mesh,
        out_type=jax.ShapeDtypeStruct((SEQ_LEN, E_DIMS), jnp.float32),
        scratch_types=[
            pltpu.VMEM((NCHUNK, CHUNK), jnp.int32),
            pltpu.VMEM((CHUNK, E_DIMS), jnp.float32),
            pltpu.SemaphoreType.DMA,
        ],
    )(_gather_kernel)
    return k(x32, E_class)


# speedup vs baseline: 1.4172x; 1.0814x over previous
import functools

import jax
import jax.numpy as jnp
from jax import lax
from jax.experimental import pallas as pl
from jax.experimental.pallas import tpu as pltpu
from jax.experimental.pallas import tpu_sc as plsc

SEQ_LEN = 16384
E_DIMS = 256
NUM_WORKERS = 32
B_PER_W = SEQ_LEN // NUM_WORKERS
CHUNK = 128
NCHUNK = B_PER_W // CHUNK


NBUF = 3


def _gather_kernel(x_hbm, tbl_hbm, out_hbm, idx_v, rows0, rows1, rows2,
                   gs0, gs1, gs2, ss0, ss1, ss2):
    rows = (rows0, rows1, rows2)
    gsem = (gs0, gs1, gs2)
    ssem = (ss0, ss1, ss2)
    wid = lax.axis_index("s") * 2 + lax.axis_index("c")
    base = wid * B_PER_W
    pltpu.sync_copy(x_hbm.at[wid], idx_v)
    gathers = [None] * NCHUNK
    stores = [None] * NCHUNK
    for j in range(min(NBUF, NCHUNK)):
        gathers[j] = pltpu.async_copy(tbl_hbm.at[idx_v.at[j]], rows[j % NBUF],
                                      gsem[j % NBUF])
    for j in range(NCHUNK):
        b = j % NBUF
        gathers[j].wait()
        stores[j] = pltpu.async_copy(rows[b],
                                     out_hbm.at[pl.ds(base + j * CHUNK, CHUNK)],
                                     ssem[b])
        if j + NBUF < NCHUNK:
            stores[j].wait()
            gathers[j + NBUF] = pltpu.async_copy(
                tbl_hbm.at[idx_v.at[j + NBUF]], rows[b], gsem[b])
    for j in range(max(0, NCHUNK - NBUF), NCHUNK):
        stores[j].wait()


@functools.partial(jax.jit, static_argnames=())
def kernel(x, E_class):
    x32 = x.astype(jnp.int32).reshape(NUM_WORKERS, NCHUNK, CHUNK)
    mesh = plsc.VectorSubcoreMesh(core_axis_name="c", subcore_axis_name="s")
    k = functools.partial(
        pl.kernel,
        mesh=mesh,
        out_type=jax.ShapeDtypeStruct((SEQ_LEN, E_DIMS), jnp.float32),
        scratch_types=[
            pltpu.VMEM((NCHUNK, CHUNK), jnp.int32),
            pltpu.VMEM((CHUNK, E_DIMS), jnp.float32),
            pltpu.VMEM((CHUNK, E_DIMS), jnp.float32),
            pltpu.VMEM((CHUNK, E_DIMS), jnp.float32),
            pltpu.SemaphoreType.DMA,
            pltpu.SemaphoreType.DMA,
            pltpu.SemaphoreType.DMA,
            pltpu.SemaphoreType.DMA,
            pltpu.SemaphoreType.DMA,
            pltpu.SemaphoreType.DMA,
        ],
    )(_gather_kernel)
    return k(x32, E_class)


# speedup vs baseline: 1.4396x; 1.0158x over previous
import functools

import jax
import jax.numpy as jnp
from jax import lax
from jax.experimental import pallas as pl
from jax.experimental.pallas import tpu as pltpu
from jax.experimental.pallas import tpu_sc as plsc

SEQ_LEN = 16384
E_DIMS = 256
NUM_WORKERS = 32
B_PER_W = SEQ_LEN // NUM_WORKERS
CHUNK = 64
NCHUNK = B_PER_W // CHUNK
NBUF = 6


def _gather_kernel(x_hbm, tbl_hbm, out_hbm, idx_v, *bufs_and_sems):
    rows = bufs_and_sems[:NBUF]
    gsem = bufs_and_sems[NBUF:2 * NBUF]
    ssem = bufs_and_sems[2 * NBUF:3 * NBUF]
    wid = lax.axis_index("s") * 2 + lax.axis_index("c")
    base = wid * B_PER_W
    pltpu.sync_copy(x_hbm.at[wid], idx_v)
    gathers = [None] * NCHUNK
    stores = [None] * NCHUNK
    for j in range(min(NBUF, NCHUNK)):
        gathers[j] = pltpu.async_copy(tbl_hbm.at[idx_v.at[j]], rows[j % NBUF],
                                      gsem[j % NBUF])
    for j in range(NCHUNK):
        b = j % NBUF
        gathers[j].wait()
        stores[j] = pltpu.async_copy(rows[b],
                                     out_hbm.at[pl.ds(base + j * CHUNK, CHUNK)],
                                     ssem[b])
        if j + NBUF < NCHUNK:
            stores[j].wait()
            gathers[j + NBUF] = pltpu.async_copy(
                tbl_hbm.at[idx_v.at[j + NBUF]], rows[b], gsem[b])
    for j in range(max(0, NCHUNK - NBUF), NCHUNK):
        stores[j].wait()


def kernel(x, E_class):
    x32 = x.astype(jnp.int32).reshape(NUM_WORKERS, NCHUNK, CHUNK)
    mesh = plsc.VectorSubcoreMesh(core_axis_name="c", subcore_axis_name="s")
    scratch = [pltpu.VMEM((NCHUNK, CHUNK), jnp.int32)]
    scratch += [pltpu.VMEM((CHUNK, E_DIMS), jnp.float32) for _ in range(NBUF)]
    scratch += [pltpu.SemaphoreType.DMA for _ in range(2 * NBUF)]
    k = functools.partial(
        pl.kernel,
        mesh=mesh,
        out_type=jax.ShapeDtypeStruct((SEQ_LEN, E_DIMS), jnp.float32),
        scratch_types=scratch,
    )(_gather_kernel)
    return k(x32, E_class)
